# Initial kernel scaffold; baseline (speedup 1.0000x reference)
import functools

import jax
import jax.numpy as jnp
from jax import lax
from jax.experimental import pallas as pl
from jax.experimental.pallas import tpu as pltpu
from jax.experimental.pallas import tpu_sc as plsc

NC = 2
NS = 16
NW = NC * NS
LANES = 16


def _make_kernel(B, L, D, V):
    assert B % NW == 0
    rows_per_w = B // NW
    chunks = []
    off = 0
    while off < L:
        n = min(128, L - off)
        chunks.append((off, n))
        off += n
    vregs_per_tok = D // LANES

    mesh = plsc.VectorSubcoreMesh(
        core_axis_name="c", subcore_axis_name="s", num_cores=NC, num_subcores=NS
    )

    @functools.partial(
        pl.kernel,
        out_type=jax.ShapeDtypeStruct((B, L, D), jnp.float32),
        mesh=mesh,
        scratch_types=[
            pltpu.VMEM((L,), jnp.int32),
            pltpu.VMEM((L,), jnp.float32),
            pltpu.VMEM((L, D), jnp.float32),
            pltpu.VMEM((L, D), jnp.float32),
            pltpu.SemaphoreType.DMA,
        ],
    )
    def k(x_hbm, conf_hbm, tab_hbm, pos_hbm, out_hbm, idx_v, conf_v, pos_v, rows_v, sem):
        wid = lax.axis_index("s") * NC + lax.axis_index("c")
        base_row = wid * rows_per_w
        pltpu.sync_copy(pos_hbm, pos_v)

        def row_body(i, _):
            row = base_row + i
            pltpu.sync_copy(x_hbm.at[row], idx_v)
            pltpu.sync_copy(conf_hbm.at[row], conf_v)
            cps = [
                pltpu.async_copy(
                    tab_hbm.at[idx_v.at[pl.ds(off, n)]],
                    rows_v.at[pl.ds(off, n)],
                    sem,
                )
                for (off, n) in chunks
            ]
            for cp in cps:
                cp.wait()

            def tok_body(t, _):
                cv = jnp.full((LANES,), conf_v[t], jnp.float32)
                for j in range(vregs_per_tok):
                    sl = pl.ds(j * LANES, LANES)
                    rows_v[t, sl] = (rows_v[t, sl] + pos_v[t, sl]) * cv
                return 0

            lax.fori_loop(0, L, tok_body, 0)
            pltpu.sync_copy(rows_v, out_hbm.at[row])
            return 0

        lax.fori_loop(0, rows_per_w, row_body, 0)

    return k


def kernel(x, MSAconf, class_embedding, pos_embedding):
    B, L = x.shape
    V, D = class_embedding.shape
    x = x.astype(jnp.int32)
    conf = MSAconf.astype(jnp.float32)
    pos = pos_embedding[:L].astype(jnp.float32)
    k = _make_kernel(B, L, D, V)
    return k(x, conf, class_embedding.astype(jnp.float32), pos)


# speedup vs baseline: 3.4572x; 3.4572x over previous
import functools

import jax
import jax.numpy as jnp
from jax import lax
from jax.experimental import pallas as pl
from jax.experimental.pallas import tpu as pltpu
from jax.experimental.pallas import tpu_sc as plsc

NC = 2
NS = 16
NW = NC * NS
LANES = 16


def _make_kernel(B, L, D, V):
    assert B % NW == 0
    rows_per_w = B // NW
    chunks = []
    off = 0
    while off < L:
        n = min(128, L - off)
        chunks.append((off, n))
        off += n
    vregs_per_tok = D // LANES

    mesh = plsc.VectorSubcoreMesh(
        core_axis_name="c", subcore_axis_name="s", num_cores=NC, num_subcores=NS
    )

    @functools.partial(
        pl.kernel,
        out_type=jax.ShapeDtypeStruct((B, L, D), jnp.float32),
        mesh=mesh,
        scratch_types=[
            pltpu.VMEM((L,), jnp.int32),
            pltpu.VMEM((L,), jnp.float32),
            pltpu.VMEM((L, D), jnp.float32),
            pltpu.VMEM((L, D), jnp.float32),
            pltpu.VMEM((L, D), jnp.float32),
            pltpu.SemaphoreType.DMA,
        ],
        compiler_params=pltpu.CompilerParams(use_tc_tiling_on_sc=False),
    )
    def k(x_hbm, conf_hbm, tab_hbm, pos_hbm, out_hbm, idx_v, conf_v, pos_v, rows_v, out_v, sem):
        wid = lax.axis_index("s") * NC + lax.axis_index("c")
        base_row = wid * rows_per_w
        pltpu.sync_copy(pos_hbm, pos_v)

        def row_body(i, _):
            row = base_row + i
            pltpu.sync_copy(x_hbm.at[row], idx_v)
            pltpu.sync_copy(conf_hbm.at[row], conf_v)
            cps = [
                pltpu.async_copy(
                    tab_hbm.at[idx_v.at[pl.ds(off, n)]],
                    rows_v.at[pl.ds(off, n)],
                    sem,
                )
                for (off, n) in chunks
            ]
            for cp in cps:
                cp.wait()

            def do_group(tb):
                cvec = conf_v[pl.ds(tb, LANES)]
                for kk in range(LANES):
                    cv = jnp.full((LANES,), cvec[kk], jnp.float32)
                    t = tb + kk
                    for j in range(vregs_per_tok):
                        sl = pl.ds(j * LANES, LANES)
                        out_v[t, sl] = (rows_v[t, sl] + pos_v[t, sl]) * cv

            def grp_body(g, _):
                do_group(g * LANES)
                return 0

            lax.fori_loop(0, L // LANES, grp_body, 0)
            if L % LANES:
                do_group(L - LANES)
            pltpu.sync_copy(out_v, out_hbm.at[row])
            return 0

        lax.fori_loop(0, rows_per_w, row_body, 0)

    return k


def kernel(x, MSAconf, class_embedding, pos_embedding):
    B, L = x.shape
    V, D = class_embedding.shape
    x = x.astype(jnp.int32)
    conf = MSAconf.astype(jnp.float32)
    pos = pos_embedding[:L].astype(jnp.float32)
    k = _make_kernel(B, L, D, V)
    return k(x, conf, class_embedding.astype(jnp.float32), pos)


# speedup vs baseline: 4.5447x; 1.3146x over previous
import functools

import jax
import jax.numpy as jnp
from jax import lax
from jax.experimental import pallas as pl
from jax.experimental.pallas import tpu as pltpu
from jax.experimental.pallas import tpu_sc as plsc

NC = 2
NS = 16
NW = NC * NS
LANES = 16
R = 4


def _make_kernel(B, L, D, V):
    assert B % NW == 0
    rows_per_w = B // NW
    assert rows_per_w % R == 0
    nblk = rows_per_w // R
    assert nblk % 2 == 0 and nblk >= 4
    chunks = []
    off = 0
    while off < L:
        n = min(128, L - off)
        chunks.append((off, n))
        off += n
    vregs_per_tok = D // LANES
    full_groups = L // LANES
    rem = L % LANES

    mesh = plsc.VectorSubcoreMesh(
        core_axis_name="c", subcore_axis_name="s", num_cores=NC, num_subcores=NS
    )

    @functools.partial(
        pl.kernel,
        out_type=jax.ShapeDtypeStruct((B, L, D), jnp.float32),
        mesh=mesh,
        scratch_types=[
            pltpu.VMEM((R, L), jnp.int32),
            pltpu.VMEM((R, L), jnp.int32),
            pltpu.VMEM((R, L), jnp.float32),
            pltpu.VMEM((R, L), jnp.float32),
            pltpu.VMEM((R, L, D), jnp.float32),
            pltpu.VMEM((R, L, D), jnp.float32),
            pltpu.VMEM((L, D), jnp.float32),
            pltpu.SemaphoreType.DMA,
            pltpu.SemaphoreType.DMA,
            pltpu.SemaphoreType.DMA,
            pltpu.SemaphoreType.DMA,
            pltpu.SemaphoreType.DMA,
            pltpu.SemaphoreType.DMA,
        ],
        compiler_params=pltpu.CompilerParams(use_tc_tiling_on_sc=False),
    )
    def k(x_hbm, conf_hbm, tab_hbm, pos_hbm, out_hbm,
          idx0, idx1, cf0, cf1, rw0, rw1, pos_v,
          si0, si1, sg0, sg1, so0, so1):
        idxs = [idx0, idx1]
        cfs = [cf0, cf1]
        rws = [rw0, rw1]
        sis = [si0, si1]
        sgs = [sg0, sg1]
        sos = [so0, so1]

        wid = lax.axis_index("s") * NC + lax.axis_index("c")
        base_row = wid * rows_per_w
        pltpu.sync_copy(pos_hbm, pos_v)

        def blk_row0(b):
            return base_row + jnp.minimum(b * R, (nblk - 1) * R)

        def fetch_start(b, s):
            r0 = blk_row0(b)
            pltpu.async_copy(x_hbm.at[pl.ds(r0, R)], idxs[s], sis[s])
            pltpu.async_copy(conf_hbm.at[pl.ds(r0, R)], cfs[s], sis[s])

        def fetch_wait(s):
            pltpu.make_async_copy(x_hbm.at[pl.ds(0, R)], idxs[s], sis[s]).wait()
            pltpu.make_async_copy(conf_hbm.at[pl.ds(0, R)], cfs[s], sis[s]).wait()

        def gather_start(b, s):
            for r in range(R):
                for (o, n) in chunks:
                    pltpu.async_copy(
                        tab_hbm.at[idxs[s].at[r, pl.ds(o, n)]],
                        rws[s].at[r, pl.ds(o, n)],
                        sgs[s],
                    )

        def gather_wait(s):
            for r in range(R):
                for (o, n) in chunks:
                    pltpu.make_async_copy(
                        tab_hbm.at[idxs[s].at[r, pl.ds(o, n)]],
                        rws[s].at[r, pl.ds(o, n)],
                        sgs[s],
                    ).wait()

        def out_start(b, s):
            pltpu.async_copy(rws[s], out_hbm.at[pl.ds(blk_row0(b), R)], sos[s])

        def out_wait(s):
            pltpu.make_async_copy(rws[s], out_hbm.at[pl.ds(0, R)], sos[s]).wait()

        def compute(s):
            rows = rws[s]
            conf = cfs[s]
            for r in range(R):
                def do_group(tb, lane_lo):
                    cvec = conf[r, pl.ds(tb, LANES)]
                    for kk in range(lane_lo, LANES):
                        cv = jnp.full((LANES,), cvec[kk], jnp.float32)
                        t = tb + kk
                        for j in range(vregs_per_tok):
                            sl = pl.ds(j * LANES, LANES)
                            rows[r, t, sl] = (rows[r, t, sl] + pos_v[t, sl]) * cv

                def grp_body(g, _):
                    do_group(g * LANES, 0)
                    return 0

                lax.fori_loop(0, full_groups, grp_body, 0)
                if rem:
                    do_group(L - LANES, LANES - rem)

        def process(b, s, drain_out):
            fetch_wait(s)
            if drain_out:
                out_wait(s)
            gather_start(b, s)
            fetch_start(b + 1, s ^ 1)
            gather_wait(s)
            compute(s)
            out_start(b, s)

        fetch_start(0, 0)
        process(0, 0, False)
        process(1, 1, False)

        def body(i, _):
            b = 2 * i
            process(b, 0, True)
            process(b + 1, 1, True)
            return 0

        lax.fori_loop(1, nblk // 2, body, 0)

        fetch_wait(nblk % 2)
        out_wait(0)
        out_wait(1)

    return k


def kernel(x, MSAconf, class_embedding, pos_embedding):
    B, L = x.shape
    V, D = class_embedding.shape
    x = x.astype(jnp.int32)
    conf = MSAconf.astype(jnp.float32)
    pos = pos_embedding[:L].astype(jnp.float32)
    k = _make_kernel(B, L, D, V)
    return k(x, conf, class_embedding.astype(jnp.float32), pos)
